# baseline (speedup 1.0000x reference)
import functools

import jax
import jax.numpy as jnp
from jax.experimental import pallas as pl
from jax.experimental.pallas import tpu as pltpu

_BQ = 128
_BK = 16384


def _patchcore_kernel(f2_ref, m_ref, qsq_ref, msq_ref,
                      min_ref, idx_ref, score_ref, *, n_chunks, k_total):
    f2 = f2_ref[...]
    q_sq = qsq_ref[...]

    def body(ki, carry):
        best_d, best_i = carry
        m = m_ref[pl.ds(ki * _BK, _BK), :]
        m_sq = msq_ref[:, pl.ds(ki * _BK, _BK)]
        cross2 = jax.lax.dot_general(
            f2, m, (((1,), (1,)), ((), ())),
            preferred_element_type=jnp.float32)
        dist = jnp.sqrt(jnp.maximum((q_sq + m_sq) + cross2, 0.0))
        bminv = jnp.min(dist, axis=1)
        bidx = jnp.argmin(dist, axis=1).astype(jnp.int32)
        bidx = bidx + ki * _BK
        take = bminv < best_d
        return (jnp.where(take, bminv, best_d),
                jnp.where(take, bidx, best_i))

    init = (jnp.full((_BQ,), jnp.inf, dtype=jnp.float32),
            jnp.zeros((_BQ,), dtype=jnp.int32))
    best_d, best_i = jax.lax.fori_loop(0, n_chunks, body, init)
    min_ref[...] = best_d
    idx_ref[...] = best_i

    block_max = jnp.max(best_d)[None, None]
    qi = pl.program_id(0)

    @pl.when(qi == 0)
    def _():
        score_ref[...] = block_max

    @pl.when(qi != 0)
    def _():
        score_ref[...] = jnp.maximum(score_ref[...], block_max)


def kernel(features, patch_memory):
    q, d = features.shape
    k, _ = patch_memory.shape
    n_chunks = k // _BK

    q_sq = jnp.sum(features * features, axis=1, keepdims=True)
    m_sq = jnp.sum(patch_memory * patch_memory, axis=1)[None, :]
    f2 = features * -2.0

    body = functools.partial(_patchcore_kernel, n_chunks=n_chunks, k_total=k)

    min_d, nn_idx, score = pl.pallas_call(
        body,
        grid=(q // _BQ,),
        in_specs=[
            pl.BlockSpec((_BQ, d), lambda qi: (qi, 0)),
            pl.BlockSpec((k, d), lambda qi: (0, 0)),
            pl.BlockSpec((_BQ, 1), lambda qi: (qi, 0)),
            pl.BlockSpec((1, k), lambda qi: (0, 0)),
        ],
        out_specs=[
            pl.BlockSpec((_BQ,), lambda qi: (qi,)),
            pl.BlockSpec((_BQ,), lambda qi: (qi,)),
            pl.BlockSpec((1, 1), lambda qi: (0, 0)),
        ],
        out_shape=[
            jax.ShapeDtypeStruct((q,), jnp.float32),
            jax.ShapeDtypeStruct((q,), jnp.int32),
            jax.ShapeDtypeStruct((1, 1), jnp.float32),
        ],
        compiler_params=pltpu.CompilerParams(
            vmem_limit_bytes=60 * 1024 * 1024,
        ),
    )(f2, patch_memory, q_sq, m_sq)
    return min_d, nn_idx, score[0, 0]


# speedup vs baseline: 1.4182x; 1.4182x over previous
import functools

import jax
import jax.numpy as jnp
from jax.experimental import pallas as pl
from jax.experimental.pallas import tpu as pltpu

_BQ = 256


def _patchcore_kernel(f2_ref, m_ref, qsq_ref, msq_ref,
                      min_ref, idx_ref, score_ref):
    f2 = f2_ref[...]
    q_sq = qsq_ref[...]
    m_sq = msq_ref[...]
    cross2 = jax.lax.dot_general(
        f2, m_ref[...], (((1,), (1,)), ((), ())),
        preferred_element_type=jnp.float32)
    dist = jnp.sqrt(jnp.maximum((q_sq + m_sq) + cross2, 0.0))
    best_d = jnp.min(dist, axis=1)
    best_i = jnp.argmin(dist, axis=1).astype(jnp.int32)
    min_ref[...] = best_d
    idx_ref[...] = best_i

    block_max = jnp.max(best_d)[None, None]
    qi = pl.program_id(0)

    @pl.when(qi == 0)
    def _():
        score_ref[...] = block_max

    @pl.when(qi != 0)
    def _():
        score_ref[...] = jnp.maximum(score_ref[...], block_max)


def kernel(features, patch_memory):
    q, d = features.shape
    k, _ = patch_memory.shape
    q_sq = jnp.sum(features * features, axis=1, keepdims=True)
    m_sq = jnp.sum(patch_memory * patch_memory, axis=1)[None, :]
    f2 = features * -2.0

    min_d, nn_idx, score = pl.pallas_call(
        _patchcore_kernel,
        grid=(q // _BQ,),
        in_specs=[
            pl.BlockSpec((_BQ, d), lambda qi: (qi, 0)),
            pl.BlockSpec((k, d), lambda qi: (0, 0)),
            pl.BlockSpec((_BQ, 1), lambda qi: (qi, 0)),
            pl.BlockSpec((1, k), lambda qi: (0, 0)),
        ],
        out_specs=[
            pl.BlockSpec((_BQ,), lambda qi: (qi,)),
            pl.BlockSpec((_BQ,), lambda qi: (qi,)),
            pl.BlockSpec((1, 1), lambda qi: (0, 0)),
        ],
        out_shape=[
            jax.ShapeDtypeStruct((q,), jnp.float32),
            jax.ShapeDtypeStruct((q,), jnp.int32),
            jax.ShapeDtypeStruct((1, 1), jnp.float32),
        ],
        compiler_params=pltpu.CompilerParams(
            vmem_limit_bytes=60 * 1024 * 1024,
        ),
    )(f2, patch_memory, q_sq, m_sq)
    return min_d, nn_idx, score[0, 0]


# speedup vs baseline: 1.6649x; 1.1740x over previous
import jax
import jax.numpy as jnp
from jax.experimental import pallas as pl
from jax.experimental.pallas import tpu as pltpu

_BQ = 256
_NC = 2

_FLT_MIN = 1.1754943508222875e-38


def _bits(x):
    return jax.lax.bitcast_convert_type(x, jnp.int32)


def _f32(x):
    return jax.lax.bitcast_convert_type(x, jnp.float32)


def _patchcore_kernel(f2_ref, m_ref, qsq_ref, msq_ref,
                      min_ref, idx_ref, score_ref, dsq_ref):
    k_total = m_ref.shape[0]
    bkc = k_total // _NC
    f2 = f2_ref[...]
    q_sq = qsq_ref[...]

    def pass1(ci, bm):
        m = m_ref[pl.ds(ci * bkc, bkc), :]
        m_sq = msq_ref[:, pl.ds(ci * bkc, bkc)]
        cross2 = jax.lax.dot_general(
            f2, m, (((1,), (1,)), ((), ())),
            preferred_element_type=jnp.float32)
        dsq = jnp.maximum((q_sq + m_sq) + cross2, 0.0)
        dsq_ref[:, pl.ds(ci * bkc, bkc)] = dsq
        return jnp.minimum(bm, jnp.min(dsq, axis=1, keepdims=True))

    bminsq = jax.lax.fori_loop(
        0, _NC, pass1, jnp.full((_BQ, 1), jnp.inf, dtype=jnp.float32))

    s = jnp.sqrt(bminsq)
    u = _f32(_bits(s) + 1)
    t = _f32(_bits(u * u) + 2)
    for _ in range(9):
        cand = _f32(_bits(t) - 1)
        t = jnp.where(jnp.sqrt(cand) >= u, cand, t)
    thresh = jnp.where(bminsq == 0.0, _FLT_MIN, t)

    def pass2(ci, bi):
        dsq = dsq_ref[:, pl.ds(ci * bkc, bkc)]
        iota = jax.lax.broadcasted_iota(jnp.int32, (_BQ, bkc), 1)
        masked = jnp.where(dsq < thresh, iota, k_total)
        lmin = jnp.min(masked, axis=1) + ci * bkc
        return jnp.minimum(bi, lmin)

    best_i = jax.lax.fori_loop(
        0, _NC, pass2, jnp.full((_BQ,), jnp.int32(2 ** 30), dtype=jnp.int32))

    min_ref[...] = s[:, 0]
    idx_ref[...] = best_i

    block_max = jnp.max(s)[None, None]
    qi = pl.program_id(0)

    @pl.when(qi == 0)
    def _():
        score_ref[...] = block_max

    @pl.when(qi != 0)
    def _():
        score_ref[...] = jnp.maximum(score_ref[...], block_max)


def kernel(features, patch_memory):
    q, d = features.shape
    k, _ = patch_memory.shape

    q_sq = jnp.sum(features * features, axis=1, keepdims=True)
    m_sq = jnp.sum(patch_memory * patch_memory, axis=1)[None, :]
    f2 = features * -2.0

    min_d, nn_idx, score = pl.pallas_call(
        _patchcore_kernel,
        grid=(q // _BQ,),
        in_specs=[
            pl.BlockSpec((_BQ, d), lambda qi: (qi, 0)),
            pl.BlockSpec((k, d), lambda qi: (0, 0)),
            pl.BlockSpec((_BQ, 1), lambda qi: (qi, 0)),
            pl.BlockSpec((1, k), lambda qi: (0, 0)),
        ],
        out_specs=[
            pl.BlockSpec((_BQ,), lambda qi: (qi,)),
            pl.BlockSpec((_BQ,), lambda qi: (qi,)),
            pl.BlockSpec((1, 1), lambda qi: (0, 0)),
        ],
        out_shape=[
            jax.ShapeDtypeStruct((q,), jnp.float32),
            jax.ShapeDtypeStruct((q,), jnp.int32),
            jax.ShapeDtypeStruct((1, 1), jnp.float32),
        ],
        scratch_shapes=[pltpu.VMEM((_BQ, k), jnp.float32)],
        compiler_params=pltpu.CompilerParams(
            vmem_limit_bytes=60 * 1024 * 1024,
        ),
    )(f2, patch_memory, q_sq, m_sq)
    return min_d, nn_idx, score[0, 0]
